# baseline (speedup 1.0000x reference)
import functools

import jax
import jax.numpy as jnp
from jax import lax
from jax.experimental import pallas as pl
from jax.experimental.pallas import tpu as pltpu
from jax.experimental.pallas import tpu_sc as plsc

EMBED = 64
NQ = 4
QW = EMBED // NQ
SEQ = 200
BATCH = 1024
NW = 32
BPW = BATCH // NW
LANES = 16
NBUF = 4
QUADS = BPW // NBUF


def _emb_body(ids_hbm, t0, t1, t2, t3, pos_hbm,
              o0, o1, o2, o3, idx_v, pos_v, bufs, gsems, ssems):
    tq = (t0, t1, t2, t3)
    oq = (o0, o1, o2, o3)
    c = lax.axis_index("c")
    s = lax.axis_index("s")
    wid = s * 2 + c
    b0 = wid * BPW

    pltpu.sync_copy(ids_hbm.at[pl.ds(b0, BPW)], idx_v)
    pltpu.sync_copy(pos_hbm.at[pl.ds(0, SEQ)], pos_v)

    def drain(sem):
        pltpu.make_async_copy(pos_hbm.at[pl.ds(0, SEQ)], pos_v, sem).wait()

    def start_gather(bl, b):
        for q in range(NQ):
            for off, n in ((0, 104), (104, 96)):
                pltpu.async_copy(
                    tq[q].at[idx_v.at[bl, pl.ds(off, n)]],
                    bufs[b][q].at[pl.ds(off, n)],
                    gsems[b],
                )

    def start_store(bl, b):
        for q in range(NQ):
            pltpu.async_copy(bufs[b][q], oq[q].at[b0 + bl], ssems[b])

    def add_pos(b):
        def add_row(j, c2):
            for jj in range(2):
                r = 2 * j + jj
                for q in range(NQ):
                    plsc.addupdate(
                        bufs[b][q].at[r, pl.ds(0, QW)],
                        pos_v[r, pl.ds(q * QW, QW)],
                    )
            return c2

        lax.fori_loop(0, SEQ // 2, add_row, 0)

    start_gather(0, 0)
    start_gather(1, 1)

    def quad(q, carry):
        for i in range(NBUF):
            bl = NBUF * q + i
            b2 = (i + 2) % NBUF
            if i < 2:
                @pl.when(q >= 1)
                def _():
                    drain(ssems[b2])
                    start_gather(bl + 2, b2)

                @pl.when(q < 1)
                def _():
                    start_gather(bl + 2, b2)
            else:
                drain(ssems[b2])

                @pl.when(q < QUADS - 1)
                def _():
                    start_gather(bl + 2, b2)

            drain(gsems[i])
            add_pos(i)
            start_store(bl, i)
        return carry

    lax.fori_loop(0, QUADS, quad, 0)
    drain(ssems[2])
    drain(ssems[3])


@jax.jit
def _emb(ids, tqs, pos):
    mesh = plsc.VectorSubcoreMesh(core_axis_name="c", subcore_axis_name="s")
    f = functools.partial(
        pl.kernel,
        mesh=mesh,
        out_type=[jax.ShapeDtypeStruct((BATCH, SEQ, QW), jnp.float32)
                  for _ in range(NQ)],
        scratch_types=[
            pltpu.VMEM((BPW, SEQ), jnp.int32),
            pltpu.VMEM((SEQ, EMBED), jnp.float32),
            [[pltpu.VMEM((SEQ, QW), jnp.float32) for _ in range(NQ)]
             for _ in range(NBUF)],
            [pltpu.SemaphoreType.DMA for _ in range(NBUF)],
            [pltpu.SemaphoreType.DMA for _ in range(NBUF)],
        ],
        compiler_params=pltpu.CompilerParams(use_tc_tiling_on_sc=False),
    )(_emb_body)
    return f(ids, *tqs, pos)


def kernel(token_ids, token_table, pos_table):
    tqs = [token_table[:, q * QW:(q + 1) * QW] for q in range(NQ)]
    outs = _emb(token_ids, tqs, pos_table)
    return jnp.concatenate(outs, axis=2)


# speedup vs baseline: 3.7939x; 3.7939x over previous
import functools

import jax
import jax.numpy as jnp
from jax import lax
from jax.experimental import pallas as pl
from jax.experimental.pallas import tpu as pltpu
from jax.experimental.pallas import tpu_sc as plsc

EMBED = 64
SEQ = 200
BATCH = 1024
NW = 32
BPW = BATCH // NW
LANES = 16
NBUF = 4
QUADS = BPW // NBUF


def _emb_body(ids_hbm, table_hbm, pos_hbm, out_hbm, idx_v, pos_v, bufs, gsems, ssems):
    c = lax.axis_index("c")
    s = lax.axis_index("s")
    wid = s * 2 + c
    b0 = wid * BPW

    pltpu.sync_copy(ids_hbm.at[pl.ds(b0, BPW)], idx_v)
    pltpu.sync_copy(pos_hbm.at[pl.ds(0, SEQ)], pos_v)

    def start_gather(bl, b):
        for off, n in ((0, 104), (104, 96)):
            pltpu.async_copy(
                table_hbm.at[idx_v.at[bl, pl.ds(off, n)]],
                bufs[b].at[pl.ds(off, n)],
                gsems[b],
            )

    def wait_gather(b):
        pltpu.make_async_copy(
            table_hbm.at[idx_v.at[0, pl.ds(0, 104)]], bufs[b], gsems[b]
        ).wait()

    def start_store(bl, b):
        pltpu.async_copy(bufs[b], out_hbm.at[b0 + bl], ssems[b])

    def wait_store(b):
        pltpu.make_async_copy(bufs[b], out_hbm.at[0], ssems[b]).wait()

    def add_pos(b):
        buf = bufs[b]

        def add_row(j, c2):
            for jj in range(2):
                for k in range(EMBED // LANES):
                    sl = pl.ds(k * LANES, LANES)
                    plsc.addupdate(buf.at[2 * j + jj, sl], pos_v[2 * j + jj, sl])
            return c2

        lax.fori_loop(0, SEQ // 2, add_row, 0)

    start_gather(0, 0)
    start_gather(1, 1)

    def quad(q, carry):
        for i in range(NBUF):
            bl = NBUF * q + i
            b2 = (i + 2) % NBUF
            if i < 2:
                @pl.when(q >= 1)
                def _():
                    wait_store(b2)
                    start_gather(bl + 2, b2)

                @pl.when(q < 1)
                def _():
                    start_gather(bl + 2, b2)
            else:
                wait_store(b2)

                @pl.when(q < QUADS - 1)
                def _():
                    start_gather(bl + 2, b2)

            wait_gather(i)
            add_pos(i)
            start_store(bl, i)
        return carry

    lax.fori_loop(0, QUADS, quad, 0)
    wait_store(2)
    wait_store(3)


@jax.jit
def _emb(ids, table, pos):
    mesh = plsc.VectorSubcoreMesh(core_axis_name="c", subcore_axis_name="s")
    f = functools.partial(
        pl.kernel,
        mesh=mesh,
        out_type=jax.ShapeDtypeStruct((BATCH, SEQ, EMBED), jnp.float32),
        scratch_types=[
            pltpu.VMEM((BPW, SEQ), jnp.int32),
            pltpu.VMEM((SEQ, EMBED), jnp.float32),
            [pltpu.VMEM((SEQ, EMBED), jnp.float32) for _ in range(NBUF)],
            [pltpu.SemaphoreType.DMA for _ in range(NBUF)],
            [pltpu.SemaphoreType.DMA for _ in range(NBUF)],
        ],
        compiler_params=pltpu.CompilerParams(use_tc_tiling_on_sc=False),
    )(_emb_body)
    return f(ids, table, pos)


def kernel(token_ids, token_table, pos_table):
    return _emb(token_ids, token_table, pos_table)
